# baseline (speedup 1.0000x reference)
import functools

import jax
import jax.numpy as jnp
from jax import lax
from jax.experimental import pallas as pl
from jax.experimental.pallas import tpu as pltpu
from jax.experimental.pallas import tpu_sc as plsc


NSLICE = 5
_CHUNK = 80


def _swish(v):
    return v * jax.nn.sigmoid(v)


def _proj_body(x_ref, wi_ref, wj_ref, p_ref, q_ref):
    xv = x_ref[...]
    p_ref[...] = jnp.dot(xv, wi_ref[...], preferred_element_type=jnp.float32)
    q_ref[...] = jnp.dot(xv, wj_ref[...], preferred_element_type=jnp.float32)


def _edge_body(g_ref, amf_ref, ea_ref, wa_ref, we_ref, w2h_ref,
               w2e_ref, bm1_ref, bm2_ref, m_ref):
    ea = ea_ref[...]
    h = (g_ref[...]
         + jnp.dot(amf_ref[...], wa_ref[...], preferred_element_type=jnp.float32)
         + jnp.dot(ea, we_ref[...], preferred_element_type=jnp.float32)
         + bm1_ref[...])
    h = _swish(h)
    m = (jnp.dot(h.astype(jnp.bfloat16), w2h_ref[...],
                 preferred_element_type=jnp.float32)
         + jnp.dot(ea, w2e_ref[...], preferred_element_type=jnp.float32)
         + bm2_ref[...])
    m_ref[...] = _swish(m)


def _node_body(x_ref, a0_ref, a1_ref, anf_ref, na_ref, wux_ref, wug_ref,
               wua_ref, wun_ref, w2h_ref, w2n_ref, bu1_ref, bu2_ref, u_ref):
    na = na_ref[...]
    agg = a0_ref[0] + a1_ref[0]
    h = (jnp.dot(x_ref[...], wux_ref[...], preferred_element_type=jnp.float32)
         + jnp.dot(agg, wug_ref[...], preferred_element_type=jnp.float32)
         + jnp.dot(anf_ref[...], wua_ref[...], preferred_element_type=jnp.float32)
         + jnp.dot(na, wun_ref[...], preferred_element_type=jnp.float32)
         + bu1_ref[...])
    h = _swish(h)
    u_ref[...] = (jnp.dot(h, w2h_ref[...], preferred_element_type=jnp.float32)
                  + jnp.dot(na, w2n_ref[...], preferred_element_type=jnp.float32)
                  + bu2_ref[...])


def _make_sc_call(Es, N_pad, H, n_cores, n_sub, gather_bases, scatter_bases,
                  chain):
    nw = n_cores * n_sub
    epw = Es // nw
    C = _CHUNK
    n_chunks = epw // C
    assert Es % nw == 0 and epw % C == 0 and n_chunks % 2 == 1
    n_pairs = (n_chunks - 1) // 2
    col_groups = H // 16
    rows_per_sub = N_pad // n_sub
    assert N_pad % (8 * n_sub) == 0 and rows_per_sub % C == 0
    mesh = plsc.VectorSubcoreMesh(core_axis_name="c", subcore_axis_name="s")

    n_g = len(gather_bases)
    n_s = len(scatter_bases)

    out_type = [jax.ShapeDtypeStruct((Es, H), jnp.float32)] * n_g
    if n_s:
        out_type = out_type + [
            jax.ShapeDtypeStruct((n_cores, N_pad, H), jnp.float32)]

    scratch = ([pltpu.VMEM((epw,), jnp.int32),
                pltpu.VMEM((epw,), jnp.int32),
                pltpu.VMEM((C, H), jnp.float32),
                pltpu.VMEM((C, H), jnp.float32),
                pltpu.VMEM((C, H), jnp.float32),
                pltpu.VMEM((C, H), jnp.float32),
                pltpu.VMEM((C,), jnp.int32),
                pltpu.VMEM((C,), jnp.int32)]
               + [pltpu.SemaphoreType.DMA for _ in range(8)])
    if n_s:
        scratch.append(pltpu.VMEM_SHARED((N_pad, H), jnp.float32))

    @functools.partial(pl.kernel, out_type=out_type, mesh=mesh,
                       scratch_types=scratch)
    def sc_call(*refs):
        pos = 0
        if n_g:
            p_hbm, q_hbm = refs[0], refs[1]
            pos = 2
        dst_hbm = refs[pos]; pos += 1
        if n_g:
            src_hbm = refs[pos]; pos += 1
        m_hbms = refs[pos:pos + n_s]; pos += n_s
        if n_s and chain:
            accin_hbm = refs[pos]; pos += 1
        g_hbms = refs[pos:pos + n_g]; pos += n_g
        if n_s:
            accout_hbm = refs[pos]; pos += 1
        (idx_d, idx_s, bufp0, bufq0, bufp1, bufq1, sidx0, sidx1,
         semp0, semq0, semp1, semq1, smi0, smm0, smi1, smm1) = \
            refs[pos:pos + 16]
        pos += 16
        if n_s:
            acc_sh = refs[pos]

        cid = lax.axis_index("c")
        sid = lax.axis_index("s")
        wbase = (sid * n_cores + cid) * epw

        for gi in range(n_g):
            ebase = gather_bases[gi]
            g_hbm = g_hbms[gi]
            pltpu.sync_copy(dst_hbm.at[pl.ds(ebase + wbase, epw)], idx_d)
            pltpu.sync_copy(src_hbm.at[pl.ds(ebase + wbase, epw)], idx_s)

            def issue(ci, bufp, bufq, semp, semq):
                o = ci * C
                pltpu.async_copy(p_hbm.at[idx_d.at[pl.ds(o, C)]], bufp, semp)
                pltpu.async_copy(q_hbm.at[idx_s.at[pl.ds(o, C)]], bufq, semq)

            def drain(ci, bufp, bufq, semp, semq, g_hbm=g_hbm):
                pltpu.make_async_copy(p_hbm.at[idx_d.at[pl.ds(0, C)]], bufp,
                                      semp).wait()
                pltpu.make_async_copy(q_hbm.at[idx_s.at[pl.ds(0, C)]], bufq,
                                      semq).wait()

                def add_row(r, carry):
                    for g in range(col_groups):
                        kk = g * 16
                        bufp[r, pl.ds(kk, 16)] = (bufp[r, pl.ds(kk, 16)]
                                                  + bufq[r, pl.ds(kk, 16)])
                    return carry

                lax.fori_loop(0, C, add_row, 0)
                pltpu.sync_copy(bufp, g_hbm.at[pl.ds(wbase + ci * C, C)])

            issue(0, bufp0, bufq0, semp0, semq0)

            def pair(j, carry):
                c1 = 2 * j + 1
                issue(c1, bufp1, bufq1, semp1, semq1)
                drain(2 * j, bufp0, bufq0, semp0, semq0)
                issue(c1 + 1, bufp0, bufq0, semp0, semq0)
                drain(c1, bufp1, bufq1, semp1, semq1)
                return carry

            lax.fori_loop(0, n_pairs, pair, 0)
            drain(n_chunks - 1, bufp0, bufq0, semp0, semq0)

        if n_s:
            r0 = sid * rows_per_sub
            if chain:
                pltpu.sync_copy(accin_hbm.at[cid, pl.ds(r0, rows_per_sub)],
                                acc_sh.at[pl.ds(r0, rows_per_sub)])
            else:
                def zero_row(r, carry):
                    for g in range(col_groups):
                        bufp0[r, pl.ds(g * 16, 16)] = jnp.zeros(
                            (16,), jnp.float32)
                    return carry

                lax.fori_loop(0, C, zero_row, 0)

                def zero_acc(t, carry):
                    pltpu.sync_copy(
                        bufp0, acc_sh.at[pl.ds(r0 + t * C, C)])
                    return carry

                lax.fori_loop(0, rows_per_sub // C, zero_acc, 0)
            plsc.subcore_barrier()

            for si in range(n_s):
                ebase = scatter_bases[si]
                m_hbm = m_hbms[si]

                def sissue(ci, sidx, mbuf, smi, smm):
                    o = ci * C
                    pltpu.async_copy(
                        dst_hbm.at[pl.ds(ebase + wbase + o, C)], sidx, smi)
                    pltpu.async_copy(m_hbm.at[pl.ds(wbase + o, C)], mbuf, smm)

                def sdrain(ci, sidx, mbuf, smi, smm, m_hbm=m_hbm):
                    pltpu.make_async_copy(
                        dst_hbm.at[pl.ds(0, C)], sidx, smi).wait()
                    pltpu.make_async_copy(
                        m_hbm.at[pl.ds(0, C)], mbuf, smm).wait()
                    pltpu.sync_copy(mbuf, acc_sh.at[sidx], add=True)

                sissue(0, sidx0, bufq0, smi0, smm0)

                def spair(j, carry):
                    c1 = 2 * j + 1
                    sissue(c1, sidx1, bufq1, smi1, smm1)
                    sdrain(2 * j, sidx0, bufq0, smi0, smm0)
                    sissue(c1 + 1, sidx0, bufq0, smi0, smm0)
                    sdrain(c1, sidx1, bufq1, smi1, smm1)
                    return carry

                lax.fori_loop(0, n_pairs, spair, 0)
                sdrain(n_chunks - 1, sidx0, bufq0, smi0, smm0)

            plsc.subcore_barrier()
            pltpu.sync_copy(acc_sh.at[pl.ds(r0, rows_per_sub)],
                            accout_hbm.at[cid, pl.ds(r0, rows_per_sub)])

    return sc_call


def kernel(x, edge_index, edge_attr, node_attr, batch,
           additional_message_features, additional_node_features,
           Wm1, bm1, Wm2, bm2, Wu1, bu1, Wu2, bu2):
    N, D = x.shape
    E = edge_index.shape[1]
    H = Wm1.shape[1]
    DE = edge_attr.shape[1]
    DAM = additional_message_features.shape[1]
    DAN = additional_node_features.shape[1]
    DNA = node_attr.shape[1]

    src = edge_index[0]
    dst = edge_index[1]

    Wi = Wm1[:D]
    Wj = Wm1[D:2 * D]
    Wa = Wm1[2 * D:2 * D + DAM]
    We = Wm1[2 * D + DAM:]
    W2h = Wm2[:H].astype(jnp.bfloat16)
    W2e = Wm2[H:]
    Wux = Wu1[:D]
    Wug = Wu1[D:D + H]
    Wua = Wu1[D + H:D + H + DAN]
    Wun = Wu1[D + H + DAN:]
    Wu2h = Wu2[:H]
    Wu2n = Wu2[H:]
    bm1r = bm1.reshape(1, H)
    bm2r = bm2.reshape(1, H)
    bu1r = bu1.reshape(1, H)
    bu2r = bu2.reshape(1, H)

    full = lambda shape: pl.BlockSpec(shape, lambda i: (0,) * len(shape))

    BN = 2000
    P, Q = pl.pallas_call(
        _proj_body,
        grid=(N // BN,),
        in_specs=[pl.BlockSpec((BN, D), lambda i: (i, 0)),
                  full((D, H)), full((D, H))],
        out_specs=[pl.BlockSpec((BN, H), lambda i: (i, 0)),
                   pl.BlockSpec((BN, H), lambda i: (i, 0))],
        out_shape=[jax.ShapeDtypeStruct((N, H), jnp.float32),
                   jax.ShapeDtypeStruct((N, H), jnp.float32)],
    )(x, Wi, Wj)

    info = plsc.get_sparse_core_info()
    n_cores, n_sub = info.num_cores, info.num_subcores

    assert E % NSLICE == 0
    Es = E // NSLICE
    BE = 6400
    assert Es % BE == 0
    N_pad = ((N + n_sub * _CHUNK - 1) // (n_sub * _CHUNK)) * (n_sub * _CHUNK)

    nblk = Es // BE

    def edge_mlp(g, s):
        return pl.pallas_call(
            _edge_body,
            grid=(nblk,),
            in_specs=[pl.BlockSpec((BE, H), lambda i: (i, 0)),
                      pl.BlockSpec((BE, DAM), lambda i, s=s: (s * nblk + i, 0)),
                      pl.BlockSpec((BE, DE), lambda i, s=s: (s * nblk + i, 0)),
                      full((DAM, H)), full((DE, H)), full((H, H)),
                      full((DE, H)), full((1, H)), full((1, H))],
            out_specs=pl.BlockSpec((BE, H), lambda i: (i, 0)),
            out_shape=jax.ShapeDtypeStruct((Es, H), jnp.float32),
            compiler_params=pltpu.CompilerParams(
                dimension_semantics=("arbitrary",)),
        )(g, additional_message_features, edge_attr,
          Wa, We, W2h, W2e, bm1r, bm2r)

    mk = functools.partial(_make_sc_call, Es, N_pad, H, n_cores, n_sub)

    def one(r):
        return r[0] if isinstance(r, (tuple, list)) else r

    g0 = one(mk([0 * Es], [], False)(P, Q, dst, src))
    g1, g2 = mk([1 * Es, 2 * Es], [], False)(P, Q, dst, src)
    m0 = edge_mlp(g0, 0)
    g3, g4 = mk([3 * Es, 4 * Es], [], False)(P, Q, dst, src)
    m1 = edge_mlp(g1, 1)
    m2 = edge_mlp(g2, 2)
    acc1 = one(mk([], [0 * Es, 1 * Es, 2 * Es], False)(dst, m0, m1, m2))
    m3 = edge_mlp(g3, 3)
    m4 = edge_mlp(g4, 4)
    acc3 = one(mk([], [3 * Es, 4 * Es], True)(dst, m3, m4, acc1))

    BU = 2000
    u = pl.pallas_call(
        _node_body,
        grid=(N // BU,),
        in_specs=[pl.BlockSpec((BU, D), lambda i: (i, 0)),
                  pl.BlockSpec((1, BU, H), lambda i: (0, i, 0)),
                  pl.BlockSpec((1, BU, H), lambda i: (1, i, 0)),
                  pl.BlockSpec((BU, DAN), lambda i: (i, 0)),
                  pl.BlockSpec((BU, DNA), lambda i: (i, 0)),
                  full((D, H)), full((H, H)), full((DAN, H)),
                  full((DNA, H)), full((H, H)), full((DNA, H)),
                  full((1, H)), full((1, H))],
        out_specs=pl.BlockSpec((BU, H), lambda i: (i, 0)),
        out_shape=jax.ShapeDtypeStruct((N, H), jnp.float32),
    )(x, acc3, acc3, additional_node_features, node_attr,
      Wux, Wug, Wua, Wun, Wu2h, Wu2n, bu1r, bu2r)
    return u


# speedup vs baseline: 1.0003x; 1.0003x over previous
import functools

import jax
import jax.numpy as jnp
from jax import lax
from jax.experimental import pallas as pl
from jax.experimental.pallas import tpu as pltpu
from jax.experimental.pallas import tpu_sc as plsc


NSLICE = 5
_CHUNK = 80


def _swish(v):
    return v * jax.nn.sigmoid(v)


def _proj_body(x_ref, wi_ref, wj_ref, p_ref, q_ref):
    xv = x_ref[...]
    p_ref[...] = jnp.dot(xv, wi_ref[...], preferred_element_type=jnp.float32)
    q_ref[...] = jnp.dot(xv, wj_ref[...], preferred_element_type=jnp.float32)


def _edge_body(g_ref, amf_ref, ea_ref, wa_ref, we_ref, w2h_ref,
               w2e_ref, bm1_ref, bm2_ref, m_ref):
    ea = ea_ref[...]
    h = (g_ref[...]
         + jnp.dot(amf_ref[...], wa_ref[...], preferred_element_type=jnp.float32)
         + jnp.dot(ea, we_ref[...], preferred_element_type=jnp.float32)
         + bm1_ref[...])
    h = _swish(h)
    m = (jnp.dot(h, w2h_ref[...], preferred_element_type=jnp.float32)
         + jnp.dot(ea, w2e_ref[...], preferred_element_type=jnp.float32)
         + bm2_ref[...])
    m_ref[...] = _swish(m)


def _node_body(x_ref, a0_ref, a1_ref, anf_ref, na_ref, wux_ref, wug_ref,
               wua_ref, wun_ref, w2h_ref, w2n_ref, bu1_ref, bu2_ref, u_ref):
    na = na_ref[...]
    agg = a0_ref[0] + a1_ref[0]
    h = (jnp.dot(x_ref[...], wux_ref[...], preferred_element_type=jnp.float32)
         + jnp.dot(agg, wug_ref[...], preferred_element_type=jnp.float32)
         + jnp.dot(anf_ref[...], wua_ref[...], preferred_element_type=jnp.float32)
         + jnp.dot(na, wun_ref[...], preferred_element_type=jnp.float32)
         + bu1_ref[...])
    h = _swish(h)
    u_ref[...] = (jnp.dot(h, w2h_ref[...], preferred_element_type=jnp.float32)
                  + jnp.dot(na, w2n_ref[...], preferred_element_type=jnp.float32)
                  + bu2_ref[...])


def _make_sc_call(Es, N_pad, H, n_cores, n_sub, gather_bases, scatter_bases,
                  chain):
    nw = n_cores * n_sub
    epw = Es // nw
    C = _CHUNK
    n_chunks = epw // C
    assert Es % nw == 0 and epw % C == 0 and n_chunks % 2 == 1
    n_pairs = (n_chunks - 1) // 2
    col_groups = H // 16
    rows_per_sub = N_pad // n_sub
    assert N_pad % (8 * n_sub) == 0 and rows_per_sub % C == 0
    mesh = plsc.VectorSubcoreMesh(core_axis_name="c", subcore_axis_name="s")

    n_g = len(gather_bases)
    n_s = len(scatter_bases)

    out_type = [jax.ShapeDtypeStruct((Es, H), jnp.float32)] * n_g
    if n_s:
        out_type = out_type + [
            jax.ShapeDtypeStruct((n_cores, N_pad, H), jnp.float32)]

    scratch = ([pltpu.VMEM((epw,), jnp.int32),
                pltpu.VMEM((epw,), jnp.int32),
                pltpu.VMEM((C, H), jnp.float32),
                pltpu.VMEM((C, H), jnp.float32),
                pltpu.VMEM((C, H), jnp.float32),
                pltpu.VMEM((C, H), jnp.float32),
                pltpu.VMEM((C,), jnp.int32),
                pltpu.VMEM((C,), jnp.int32)]
               + [pltpu.SemaphoreType.DMA for _ in range(8)])
    if n_s:
        scratch.append(pltpu.VMEM_SHARED((N_pad, H), jnp.float32))

    @functools.partial(pl.kernel, out_type=out_type, mesh=mesh,
                       scratch_types=scratch)
    def sc_call(*refs):
        pos = 0
        if n_g:
            p_hbm, q_hbm = refs[0], refs[1]
            pos = 2
        dst_hbm = refs[pos]; pos += 1
        if n_g:
            src_hbm = refs[pos]; pos += 1
        m_hbms = refs[pos:pos + n_s]; pos += n_s
        if n_s and chain:
            accin_hbm = refs[pos]; pos += 1
        g_hbms = refs[pos:pos + n_g]; pos += n_g
        if n_s:
            accout_hbm = refs[pos]; pos += 1
        (idx_d, idx_s, bufp0, bufq0, bufp1, bufq1, sidx0, sidx1,
         semp0, semq0, semp1, semq1, smi0, smm0, smi1, smm1) = \
            refs[pos:pos + 16]
        pos += 16
        if n_s:
            acc_sh = refs[pos]

        cid = lax.axis_index("c")
        sid = lax.axis_index("s")
        wbase = (sid * n_cores + cid) * epw

        for gi in range(n_g):
            ebase = gather_bases[gi]
            g_hbm = g_hbms[gi]
            pltpu.sync_copy(dst_hbm.at[pl.ds(ebase + wbase, epw)], idx_d)
            pltpu.sync_copy(src_hbm.at[pl.ds(ebase + wbase, epw)], idx_s)

            def issue(ci, bufp, bufq, semp, semq):
                o = ci * C
                pltpu.async_copy(p_hbm.at[idx_d.at[pl.ds(o, C)]], bufp, semp)
                pltpu.async_copy(q_hbm.at[idx_s.at[pl.ds(o, C)]], bufq, semq)

            def drain(ci, bufp, bufq, semp, semq, g_hbm=g_hbm):
                pltpu.make_async_copy(p_hbm.at[idx_d.at[pl.ds(0, C)]], bufp,
                                      semp).wait()
                pltpu.make_async_copy(q_hbm.at[idx_s.at[pl.ds(0, C)]], bufq,
                                      semq).wait()

                def add_row(r, carry):
                    for g in range(col_groups):
                        kk = g * 16
                        bufp[r, pl.ds(kk, 16)] = (bufp[r, pl.ds(kk, 16)]
                                                  + bufq[r, pl.ds(kk, 16)])
                    return carry

                lax.fori_loop(0, C, add_row, 0)
                pltpu.sync_copy(bufp, g_hbm.at[pl.ds(wbase + ci * C, C)])

            issue(0, bufp0, bufq0, semp0, semq0)

            def pair(j, carry):
                c1 = 2 * j + 1
                issue(c1, bufp1, bufq1, semp1, semq1)
                drain(2 * j, bufp0, bufq0, semp0, semq0)
                issue(c1 + 1, bufp0, bufq0, semp0, semq0)
                drain(c1, bufp1, bufq1, semp1, semq1)
                return carry

            lax.fori_loop(0, n_pairs, pair, 0)
            drain(n_chunks - 1, bufp0, bufq0, semp0, semq0)

        if n_s:
            r0 = sid * rows_per_sub
            if chain:
                pltpu.sync_copy(accin_hbm.at[cid, pl.ds(r0, rows_per_sub)],
                                acc_sh.at[pl.ds(r0, rows_per_sub)])
            else:
                def zero_row(r, carry):
                    for g in range(col_groups):
                        bufp0[r, pl.ds(g * 16, 16)] = jnp.zeros(
                            (16,), jnp.float32)
                    return carry

                lax.fori_loop(0, C, zero_row, 0)

                def zero_acc(t, carry):
                    pltpu.sync_copy(
                        bufp0, acc_sh.at[pl.ds(r0 + t * C, C)])
                    return carry

                lax.fori_loop(0, rows_per_sub // C, zero_acc, 0)
            plsc.subcore_barrier()

            for si in range(n_s):
                ebase = scatter_bases[si]
                m_hbm = m_hbms[si]

                def sissue(ci, sidx, mbuf, smi, smm):
                    o = ci * C
                    pltpu.async_copy(
                        dst_hbm.at[pl.ds(ebase + wbase + o, C)], sidx, smi)
                    pltpu.async_copy(m_hbm.at[pl.ds(wbase + o, C)], mbuf, smm)

                def sdrain(ci, sidx, mbuf, smi, smm, m_hbm=m_hbm):
                    pltpu.make_async_copy(
                        dst_hbm.at[pl.ds(0, C)], sidx, smi).wait()
                    pltpu.make_async_copy(
                        m_hbm.at[pl.ds(0, C)], mbuf, smm).wait()
                    pltpu.sync_copy(mbuf, acc_sh.at[sidx], add=True)

                sissue(0, sidx0, bufq0, smi0, smm0)

                def spair(j, carry):
                    c1 = 2 * j + 1
                    sissue(c1, sidx1, bufq1, smi1, smm1)
                    sdrain(2 * j, sidx0, bufq0, smi0, smm0)
                    sissue(c1 + 1, sidx0, bufq0, smi0, smm0)
                    sdrain(c1, sidx1, bufq1, smi1, smm1)
                    return carry

                lax.fori_loop(0, n_pairs, spair, 0)
                sdrain(n_chunks - 1, sidx0, bufq0, smi0, smm0)

            plsc.subcore_barrier()
            pltpu.sync_copy(acc_sh.at[pl.ds(r0, rows_per_sub)],
                            accout_hbm.at[cid, pl.ds(r0, rows_per_sub)])

    return sc_call


def kernel(x, edge_index, edge_attr, node_attr, batch,
           additional_message_features, additional_node_features,
           Wm1, bm1, Wm2, bm2, Wu1, bu1, Wu2, bu2):
    N, D = x.shape
    E = edge_index.shape[1]
    H = Wm1.shape[1]
    DE = edge_attr.shape[1]
    DAM = additional_message_features.shape[1]
    DAN = additional_node_features.shape[1]
    DNA = node_attr.shape[1]

    src = edge_index[0]
    dst = edge_index[1]

    Wi = Wm1[:D]
    Wj = Wm1[D:2 * D]
    Wa = Wm1[2 * D:2 * D + DAM]
    We = Wm1[2 * D + DAM:]
    W2h = Wm2[:H]
    W2e = Wm2[H:]
    Wux = Wu1[:D]
    Wug = Wu1[D:D + H]
    Wua = Wu1[D + H:D + H + DAN]
    Wun = Wu1[D + H + DAN:]
    Wu2h = Wu2[:H]
    Wu2n = Wu2[H:]
    bm1r = bm1.reshape(1, H)
    bm2r = bm2.reshape(1, H)
    bu1r = bu1.reshape(1, H)
    bu2r = bu2.reshape(1, H)

    full = lambda shape: pl.BlockSpec(shape, lambda i: (0,) * len(shape))

    BN = 2000
    P, Q = pl.pallas_call(
        _proj_body,
        grid=(N // BN,),
        in_specs=[pl.BlockSpec((BN, D), lambda i: (i, 0)),
                  full((D, H)), full((D, H))],
        out_specs=[pl.BlockSpec((BN, H), lambda i: (i, 0)),
                   pl.BlockSpec((BN, H), lambda i: (i, 0))],
        out_shape=[jax.ShapeDtypeStruct((N, H), jnp.float32),
                   jax.ShapeDtypeStruct((N, H), jnp.float32)],
    )(x, Wi, Wj)

    info = plsc.get_sparse_core_info()
    n_cores, n_sub = info.num_cores, info.num_subcores

    assert E % NSLICE == 0
    Es = E // NSLICE
    BE = 6400
    assert Es % BE == 0
    N_pad = ((N + n_sub * _CHUNK - 1) // (n_sub * _CHUNK)) * (n_sub * _CHUNK)

    nblk = Es // BE

    def edge_mlp(g, s):
        return pl.pallas_call(
            _edge_body,
            grid=(nblk,),
            in_specs=[pl.BlockSpec((BE, H), lambda i: (i, 0)),
                      pl.BlockSpec((BE, DAM), lambda i, s=s: (s * nblk + i, 0)),
                      pl.BlockSpec((BE, DE), lambda i, s=s: (s * nblk + i, 0)),
                      full((DAM, H)), full((DE, H)), full((H, H)),
                      full((DE, H)), full((1, H)), full((1, H))],
            out_specs=pl.BlockSpec((BE, H), lambda i: (i, 0)),
            out_shape=jax.ShapeDtypeStruct((Es, H), jnp.float32),
            compiler_params=pltpu.CompilerParams(
                dimension_semantics=("arbitrary",)),
        )(g, additional_message_features, edge_attr,
          Wa, We, W2h, W2e, bm1r, bm2r)

    mk = functools.partial(_make_sc_call, Es, N_pad, H, n_cores, n_sub)

    def one(r):
        return r[0] if isinstance(r, (tuple, list)) else r

    g0 = one(mk([0 * Es], [], False)(P, Q, dst, src))
    g1, g2 = mk([1 * Es, 2 * Es], [], False)(P, Q, dst, src)
    m0 = edge_mlp(g0, 0)
    g3, g4 = mk([3 * Es, 4 * Es], [], False)(P, Q, dst, src)
    m1 = edge_mlp(g1, 1)
    m2 = edge_mlp(g2, 2)
    acc1 = one(mk([], [0 * Es, 1 * Es, 2 * Es], False)(dst, m0, m1, m2))
    m3 = edge_mlp(g3, 3)
    m4 = edge_mlp(g4, 4)
    acc3 = one(mk([], [3 * Es, 4 * Es], True)(dst, m3, m4, acc1))

    BU = 2000
    u = pl.pallas_call(
        _node_body,
        grid=(N // BU,),
        in_specs=[pl.BlockSpec((BU, D), lambda i: (i, 0)),
                  pl.BlockSpec((1, BU, H), lambda i: (0, i, 0)),
                  pl.BlockSpec((1, BU, H), lambda i: (1, i, 0)),
                  pl.BlockSpec((BU, DAN), lambda i: (i, 0)),
                  pl.BlockSpec((BU, DNA), lambda i: (i, 0)),
                  full((D, H)), full((H, H)), full((DAN, H)),
                  full((DNA, H)), full((H, H)), full((DNA, H)),
                  full((1, H)), full((1, H))],
        out_specs=pl.BlockSpec((BU, H), lambda i: (i, 0)),
        out_shape=jax.ShapeDtypeStruct((N, H), jnp.float32),
    )(x, acc3, acc3, additional_node_features, node_attr,
      Wux, Wug, Wua, Wun, Wu2h, Wu2n, bu1r, bu2r)
    return u
